# baseline (speedup 1.0000x reference)
import functools

import jax
import jax.numpy as jnp
from jax import lax
from jax.experimental import pallas as pl
from jax.experimental.pallas import tpu as pltpu
from jax.experimental.pallas import tpu_sc as plsc

N_STATES = 4096
N_RULES = 1638400
ADJ_WORDS = N_STATES * N_STATES

NUM_SC = 2
NUM_TILES = 16
LANES = 16

W_PASS = 1507328
N_PASS = -(-ADJ_WORDS // W_PASS)

E_TILE = N_RULES // NUM_TILES
BE = 2048
N_PAIR = E_TILE // (2 * BE)
VPB = BE // LANES

ZBUF = 4096


NBLK = BE // 256


def _scatter_body(body_hbm, head_hbm, w_hbm, adj_out, acc, bvec, hvec, wvec,
                  cidx, cw, lbuf, offs, cum,
                  ci0, ci1, ci2, ci3, ci4, ci5, ci6, ci7,
                  cv0, cv1, cv2, cv3, cv4, cv5, cv6, cv7,
                  zbuf, stage, ld_sem, st_sem):
    ck_i = [ci0, ci1, ci2, ci3, ci4, ci5, ci6, ci7]
    ck_w = [cv0, cv1, cv2, cv3, cv4, cv5, cv6, cv7]
    c = lax.axis_index("c")
    s = lax.axis_index("s")
    iota = lax.iota(jnp.int32, LANES)

    def _z(i, carry):
        zbuf[pl.ds(i * LANES, LANES)] = jnp.zeros((LANES,), jnp.float32)
        return carry
    lax.fori_loop(0, ZBUF // LANES, _z, 0)

    def _ld(j, p):
        eb = s * E_TILE + j * BE
        pltpu.async_copy(body_hbm.at[pl.ds(eb, BE)], bvec.at[p], ld_sem.at[p])
        pltpu.async_copy(head_hbm.at[pl.ds(eb, BE)], hvec.at[p], ld_sem.at[p])
        pltpu.async_copy(w_hbm.at[pl.ds(eb, BE)], wvec.at[p], ld_sem.at[p])

    def _ld_wait(j, p):
        eb = s * E_TILE + j * BE
        pltpu.make_async_copy(body_hbm.at[pl.ds(eb, BE)], bvec.at[p], ld_sem.at[p]).wait()
        pltpu.make_async_copy(head_hbm.at[pl.ds(eb, BE)], hvec.at[p], ld_sem.at[p]).wait()
        pltpu.make_async_copy(w_hbm.at[pl.ds(eb, BE)], wvec.at[p], ld_sem.at[p]).wait()

    def pass_body(k, carry):
        base = (k * NUM_SC + c) * W_PASS
        n_words = jnp.minimum(W_PASS, ADJ_WORDS - base)
        tile_share = n_words // NUM_TILES
        n_slabs = tile_share // ZBUF

        def _zero(j, carry2):
            off = pl.multiple_of(s * tile_share + j * ZBUF, ZBUF)
            pltpu.sync_copy(zbuf, acc.at[pl.ds(off, ZBUF)])
            return carry2
        lax.fori_loop(0, n_slabs, _zero, 0)
        plsc.subcore_barrier()

        def _do_batch(p, prev_cnt):
            one = jnp.ones((LANES,), jnp.int32)
            zero = jnp.zeros((LANES,), jnp.int32)
            sentinel = jnp.full((LANES,), 2**30, jnp.int32)
            cnt = jnp.int32(0)
            for v in range(VPB):
                off = v * LANES
                b16 = bvec[p, pl.ds(off, LANES)]
                h16 = hvec[p, pl.ds(off, LANES)]
                w16 = wvec[p, pl.ds(off, LANES)]
                local = (b16 << 12) + h16 - base
                inb = plsc.bitcast(local, jnp.uint32) < jnp.uint32(W_PASS)
                key = jnp.where(inb, local, sentinel)
                wsel = jnp.where(inb, w16, 0.0)
                sk, sv = plsc.sort_key_val(key, wsel)
                cidx[pl.ds(cnt, LANES)] = sk
                cw[pl.ds(cnt, LANES)] = sv
                t = jnp.where(inb, one, zero)
                for sh in (8, 4, 2, 1):
                    t = t + t.at[iota ^ sh].get(
                        mode="promise_in_bounds", unique_indices=True)
                cnt = cnt + t[0]
            for t in range(LANES):
                cidx[pl.ds(cnt + t * LANES, LANES)] = iota + t * LANES
                cw[pl.ds(cnt + t * LANES, LANES)] = jnp.zeros((LANES,),
                                                              jnp.float32)
            for g in range(NBLK):
                @pl.when(g * 256 < prev_cnt)
                def _():
                    pltpu.make_async_copy(ck_w[g], acc.at[ck_i[g]],
                                          st_sem).wait()
            for g in range(NBLK):
                @pl.when(g * 256 < cnt)
                def _():
                    for t in range(LANES):
                        ck_i[g][pl.ds(t * LANES, LANES)] = (
                            cidx[pl.ds(g * 256 + t * LANES, LANES)])
                        ck_w[g][pl.ds(t * LANES, LANES)] = (
                            cw[pl.ds(g * 256 + t * LANES, LANES)])
                    pltpu.async_copy(ck_w[g], acc.at[ck_i[g]], st_sem,
                                     add=True)
            return cnt

        _ld(0, 0)

        def pair_body(i, prev_cnt):
            _ld_wait(2 * i, 0)
            _ld(2 * i + 1, 1)
            c0 = _do_batch(0, prev_cnt)

            _ld_wait(2 * i + 1, 1)

            @pl.when(i < N_PAIR - 1)
            def _():
                _ld(2 * i + 2, 0)
            c1 = _do_batch(1, c0)
            return c1
        prev = lax.fori_loop(0, N_PAIR, pair_body, jnp.int32(0))
        for g in range(NBLK):
            @pl.when(g * 256 < prev)
            def _():
                pltpu.make_async_copy(ck_w[g], acc.at[ck_i[g]], st_sem).wait()
        plsc.subcore_barrier()

        def _wout(j, carry2):
            off = pl.multiple_of(s * tile_share + j * ZBUF, ZBUF)
            row = (base + off) >> 12
            pltpu.sync_copy(acc.at[pl.ds(off, ZBUF)], stage)
            pltpu.sync_copy(stage, adj_out.at[row])
            return carry2
        lax.fori_loop(0, n_slabs, _wout, 0)
        plsc.subcore_barrier()
        return carry
    n_my_passes = (N_PASS + 1 - c) // 2
    lax.fori_loop(0, n_my_passes, pass_body, 0)


_scatter_sc = functools.partial(
    pl.kernel,
    out_type=jax.ShapeDtypeStruct((N_STATES, N_STATES), jnp.float32),
    mesh=plsc.VectorSubcoreMesh(core_axis_name="c", subcore_axis_name="s"),
    compiler_params=pltpu.CompilerParams(needs_layout_passes=False),
    scratch_types=[
        pltpu.VMEM_SHARED((W_PASS,), jnp.float32),
        pltpu.VMEM((2, BE), jnp.int32),
        pltpu.VMEM((2, BE), jnp.int32),
        pltpu.VMEM((2, BE), jnp.float32),
        pltpu.VMEM((BE + 256,), jnp.int32),
        pltpu.VMEM((BE + 256,), jnp.float32),
        pltpu.VMEM((256,), jnp.int32),
        pltpu.VMEM((LANES,), jnp.int32),
        pltpu.VMEM((LANES,), jnp.int32),
    ] + [pltpu.VMEM((256,), jnp.int32)] * 8
      + [pltpu.VMEM((256,), jnp.float32)] * 8
      + [
        pltpu.VMEM((ZBUF,), jnp.float32),
        pltpu.VMEM((ZBUF,), jnp.float32),
        pltpu.SemaphoreType.DMA((2,)),
        pltpu.SemaphoreType.DMA,
    ],
)(_scatter_body)


def _mm_body(x_ref, a_ref, o_ref):
    a = a_ref[...]
    hi = a.astype(jnp.bfloat16)
    lo = (a - hi.astype(jnp.float32)).astype(jnp.bfloat16)
    xb = x_ref[...]
    acc = jnp.dot(xb, hi, preferred_element_type=jnp.float32)
    acc = acc + jnp.dot(xb, lo, preferred_element_type=jnp.float32)
    o_ref[...] = jax.nn.sigmoid(acc)


BM = 256
BN = 512


def _matmul_tc(x, adj):
    m = x.shape[0]
    return pl.pallas_call(
        _mm_body,
        grid=(N_STATES // BN, m // BM),
        in_specs=[
            pl.BlockSpec((BM, N_STATES), lambda j, i: (i, 0)),
            pl.BlockSpec((N_STATES, BN), lambda j, i: (0, j)),
        ],
        out_specs=pl.BlockSpec((BM, BN), lambda j, i: (i, j)),
        out_shape=jax.ShapeDtypeStruct((m, N_STATES), jnp.float32),
    )(x, adj)


def kernel(x, rule_indices, rule_weights):
    body = rule_indices[0]
    head = rule_indices[1]
    adj = _scatter_sc(body, head, rule_weights)
    return _matmul_tc(x.astype(jnp.bfloat16), adj)


# speedup vs baseline: 3.1984x; 3.1984x over previous
import functools

import jax
import jax.numpy as jnp
from jax import lax
from jax.experimental import pallas as pl
from jax.experimental.pallas import tpu as pltpu
from jax.experimental.pallas import tpu_sc as plsc

N_STATES = 4096
N_RULES = 1638400
ADJ_WORDS = N_STATES * N_STATES

NUM_SC = 2
NUM_TILES = 16
LANES = 16

W_PASS = 1507328
N_PASS = -(-ADJ_WORDS // W_PASS)

E_TILE = N_RULES // NUM_TILES
BE = 2048
N_PAIR = E_TILE // (2 * BE)
VPB = BE // LANES

ZBUF = 4096


def _scatter_body(body_hbm, head_hbm, w_hbm, adj_out, acc, bvec, hvec, wvec,
                  idx_a, idx_b, w_a, w_b, zbuf, stage, ld_sem, st_sem):
    c = lax.axis_index("c")
    s = lax.axis_index("s")
    iota = lax.iota(jnp.int32, LANES)

    def _z(i, carry):
        zbuf[pl.ds(i * LANES, LANES)] = jnp.zeros((LANES,), jnp.float32)
        return carry
    lax.fori_loop(0, ZBUF // LANES, _z, 0)

    def _ld(j, p):
        eb = s * E_TILE + j * BE
        pltpu.async_copy(body_hbm.at[pl.ds(eb, BE)], bvec.at[p], ld_sem.at[p])
        pltpu.async_copy(head_hbm.at[pl.ds(eb, BE)], hvec.at[p], ld_sem.at[p])
        pltpu.async_copy(w_hbm.at[pl.ds(eb, BE)], wvec.at[p], ld_sem.at[p])

    def _ld_wait(j, p):
        eb = s * E_TILE + j * BE
        pltpu.make_async_copy(body_hbm.at[pl.ds(eb, BE)], bvec.at[p], ld_sem.at[p]).wait()
        pltpu.make_async_copy(head_hbm.at[pl.ds(eb, BE)], hvec.at[p], ld_sem.at[p]).wait()
        pltpu.make_async_copy(w_hbm.at[pl.ds(eb, BE)], wvec.at[p], ld_sem.at[p]).wait()

    def pass_body(k, carry):
        base = (k * NUM_SC + c) * W_PASS
        n_words = jnp.minimum(W_PASS, ADJ_WORDS - base)
        tile_share = n_words // NUM_TILES
        n_slabs = tile_share // ZBUF

        def _zero(j, carry2):
            off = pl.multiple_of(s * tile_share + j * ZBUF, ZBUF)
            pltpu.sync_copy(zbuf, acc.at[pl.ds(off, ZBUF)])
            return carry2
        lax.fori_loop(0, n_slabs, _zero, 0)
        plsc.subcore_barrier()

        def _compute(j, p):
            for v in range(VPB):
                off = v * LANES
                b16 = bvec[p, pl.ds(off, LANES)]
                h16 = hvec[p, pl.ds(off, LANES)]
                w16 = wvec[p, pl.ds(off, LANES)]
                local = (b16 << 12) + h16 - base
                inb = plsc.bitcast(local, jnp.uint32) < jnp.uint32(W_PASS)
                dummy = iota + off
                idxbuf = idx_a if p == 0 else idx_b
                wbuf = w_a if p == 0 else w_b
                idxbuf[pl.ds(off, LANES)] = jnp.where(inb, local, dummy)
                wbuf[pl.ds(off, LANES)] = jnp.where(inb, w16, 0.0)

        def _scat_start(p):
            idxbuf = idx_a if p == 0 else idx_b
            wbuf = w_a if p == 0 else w_b
            pltpu.async_copy(wbuf, acc.at[idxbuf], st_sem.at[p], add=True)

        def _scat_wait(p):
            idxbuf = idx_a if p == 0 else idx_b
            wbuf = w_a if p == 0 else w_b
            pltpu.make_async_copy(wbuf, acc.at[idxbuf], st_sem.at[p]).wait()

        _ld(0, 0)

        def pair_body(i, carry2):
            _ld_wait(2 * i, 0)
            _ld(2 * i + 1, 1)

            @pl.when(i > 0)
            def _():
                _scat_wait(0)
            _compute(2 * i, 0)
            _scat_start(0)

            _ld_wait(2 * i + 1, 1)

            @pl.when(i < N_PAIR - 1)
            def _():
                _ld(2 * i + 2, 0)

            @pl.when(i > 0)
            def _():
                _scat_wait(1)
            _compute(2 * i + 1, 1)
            _scat_start(1)
            return carry2
        lax.fori_loop(0, N_PAIR, pair_body, 0)
        _scat_wait(0)
        _scat_wait(1)
        plsc.subcore_barrier()

        def _wout(j, carry2):
            off = pl.multiple_of(s * tile_share + j * ZBUF, ZBUF)
            row = (base + off) >> 12
            pltpu.sync_copy(acc.at[pl.ds(off, ZBUF)], stage)
            pltpu.sync_copy(stage, adj_out.at[row])
            return carry2
        lax.fori_loop(0, n_slabs, _wout, 0)
        plsc.subcore_barrier()
        return carry
    n_my_passes = (N_PASS + 1 - c) // 2
    lax.fori_loop(0, n_my_passes, pass_body, 0)


_scatter_sc = functools.partial(
    pl.kernel,
    out_type=jax.ShapeDtypeStruct((N_STATES, N_STATES), jnp.float32),
    mesh=plsc.VectorSubcoreMesh(core_axis_name="c", subcore_axis_name="s"),
    scratch_types=[
        pltpu.VMEM_SHARED((W_PASS,), jnp.float32),
        pltpu.VMEM((2, BE), jnp.int32),
        pltpu.VMEM((2, BE), jnp.int32),
        pltpu.VMEM((2, BE), jnp.float32),
        pltpu.VMEM((BE,), jnp.int32),
        pltpu.VMEM((BE,), jnp.int32),
        pltpu.VMEM((BE,), jnp.float32),
        pltpu.VMEM((BE,), jnp.float32),
        pltpu.VMEM((ZBUF,), jnp.float32),
        pltpu.VMEM((ZBUF,), jnp.float32),
        pltpu.SemaphoreType.DMA((2,)),
        pltpu.SemaphoreType.DMA((2,)),
    ],
)(_scatter_body)


def _mm_body(x_ref, a_ref, o_ref):
    a = a_ref[...]
    hi = a.astype(jnp.bfloat16)
    lo = (a - hi.astype(jnp.float32)).astype(jnp.bfloat16)
    xb = x_ref[...]
    acc = jnp.dot(xb, hi, preferred_element_type=jnp.float32)
    acc = acc + jnp.dot(xb, lo, preferred_element_type=jnp.float32)
    o_ref[...] = jax.nn.sigmoid(acc)


BN = 512


def _matmul_tc(x, adj):
    m = x.shape[0]
    return pl.pallas_call(
        _mm_body,
        grid=(N_STATES // BN,),
        in_specs=[
            pl.BlockSpec((m, N_STATES), lambda j: (0, 0)),
            pl.BlockSpec((N_STATES, BN), lambda j: (0, j)),
        ],
        out_specs=pl.BlockSpec((m, BN), lambda j: (0, j)),
        out_shape=jax.ShapeDtypeStruct((m, N_STATES), jnp.float32),
    )(x, adj)


def kernel(x, rule_indices, rule_weights):
    body = rule_indices[0]
    head = rule_indices[1]
    adj = _scatter_sc(body, head, rule_weights)
    return _matmul_tc(x.astype(jnp.bfloat16), adj)
